# Initial kernel scaffold; baseline (speedup 1.0000x reference)
import jax
import jax.numpy as jnp
from jax.experimental import pallas as pl


def kernel(users_emb, items_emb, Wpl1, Wpr1, bpr1, Wnl1, Wnr1, bnr1, Wpl2, Wpr2, bpr2, Wnl2, Wnr2, bnr2, pos_edge_index, neg_edge_index):
    raise NotImplementedError("write your pallas kernel here")


# speedup vs baseline: 1.1872x; 1.1872x over previous
import functools

import jax
import jax.numpy as jnp
from jax.experimental import pallas as pl
from jax.experimental.pallas import tpu as pltpu

N = 50000
H = 128
BN = 2000


def _dense_body(sp_ref, sn_ref, x_ref, cp_ref, cn_ref, u_ref, v_ref, w_ref,
                b_ref, o_ref):
    rp = 1.0 / jnp.clip(cp_ref[...], 1.0, None)
    rn = 1.0 / jnp.clip(cn_ref[...], 1.0, None)
    sp = sp_ref[...] * rp
    sn = sn_ref[...] * rn
    acc = jnp.dot(sp, u_ref[...], preferred_element_type=jnp.float32)
    acc = acc + jnp.dot(sn, v_ref[...], preferred_element_type=jnp.float32)
    acc = acc + jnp.dot(x_ref[...], w_ref[...],
                        preferred_element_type=jnp.float32)
    o_ref[...] = jnp.maximum(acc + b_ref[...], 0.0)


def _dense_layer(sp, sn, x, cp, cn, u, v, w, b):
    bs = pl.BlockSpec((BN, H), lambda i: (i, 0))
    cs = pl.BlockSpec((BN, 1), lambda i: (i, 0))
    ws = pl.BlockSpec((H, H), lambda i: (0, 0))
    return pl.pallas_call(
        _dense_body,
        grid=(N // BN,),
        in_specs=[bs, bs, bs, cs, cs, ws, ws, ws,
                  pl.BlockSpec((1, H), lambda i: (0, 0))],
        out_specs=bs,
        out_shape=jax.ShapeDtypeStruct((N, H), jnp.float32),
    )(sp, sn, x, cp, cn, u, v, w, b)


def _seg_sum(feat, src, dst):
    s = jax.ops.segment_sum(feat[src], dst, num_segments=N)
    return s


def kernel(users_emb, items_emb, Wpl1, Wpr1, bpr1, Wnl1, Wnr1, bnr1,
           Wpl2, Wpr2, bpr2, Wnl2, Wnr2, bnr2,
           pos_edge_index, neg_edge_index):
    x = jnp.concatenate([users_emb, items_emb], axis=0)
    ps = pos_edge_index[0].astype(jnp.int32)
    pd = pos_edge_index[1].astype(jnp.int32)
    ns = neg_edge_index[0].astype(jnp.int32)
    nd = neg_edge_index[1].astype(jnp.int32)

    ones_p = jnp.ones((ps.shape[0],), jnp.float32)
    cp = jax.ops.segment_sum(ones_p, pd, num_segments=N)[:, None]
    cn = jax.ops.segment_sum(ones_p, nd, num_segments=N)[:, None]

    z64 = jnp.zeros((H // 2, H // 2), jnp.float32)
    u1 = jnp.concatenate([Wpl1, jnp.zeros_like(Wpl1)], axis=1)
    v1 = jnp.concatenate([jnp.zeros_like(Wnl1), Wnl1], axis=1)
    w1 = jnp.concatenate([Wpr1, Wnr1], axis=1)
    b1 = jnp.concatenate([bpr1, bnr1])[None, :]

    u2 = jnp.block([[Wpl2[:64], z64], [z64, Wnl2[:64]]])
    v2 = jnp.block([[z64, Wnl2[64:]], [Wpl2[64:], z64]])
    w2 = jnp.block([[Wpr2, z64], [z64, Wnr2]])
    b2 = jnp.concatenate([bpr2, bnr2])[None, :]

    sp1 = _seg_sum(x, ps, pd)
    sn1 = _seg_sum(x, ns, nd)
    z = _dense_layer(sp1, sn1, x, cp, cn, u1, v1, w1, b1)

    sp2 = _seg_sum(z, ps, pd)
    sn2 = _seg_sum(z, ns, nd)
    out = _dense_layer(sp2, sn2, z, cp, cn, u2, v2, w2, b2)
    return out


# speedup vs baseline: 2.5710x; 2.1655x over previous
import functools

import jax
import jax.numpy as jnp
from jax import lax
from jax.experimental import pallas as pl
from jax.experimental.pallas import tpu as pltpu
from jax.experimental.pallas import tpu_sc as plsc

N = 50000
H = 128
BN = 2000
NT = 16
NPAD = 51200
RPT = NPAD // NT
CH = 16
NCH = 8
EB = 128
E = 400000
EPT = 25088
E_PAD = EPT * NT
NB = EPT // EB
DR = 1600


def _fill16(ref, n, val):
    v = jnp.full((16,), val, jnp.float32)

    def body(i, _):
        ref[pl.ds(i * 16, 16)] = v
        return 0

    lax.fori_loop(0, n // 16, body, 0)


def _gidx(bufi, b, off, src_v, gidx_v):
    for k in range(EB // 16):
        sv = src_v[pl.ds(b * EB + 16 * k, 16)]
        gidx_v[bufi, pl.ds(16 * k, 16)] = sv + off


def _scatter_job(tbl, src_v, dst_v, rows_v, gidx_v, acc, off, sem0, sem1):
    _gidx(0, 0, off, src_v, gidx_v)
    pltpu.async_copy(tbl.at[gidx_v.at[0]], rows_v.at[0], sem0)

    def body(k, _):
        b0 = 2 * k
        _gidx(1, b0 + 1, off, src_v, gidx_v)
        pltpu.async_copy(tbl.at[gidx_v.at[1]], rows_v.at[1], sem1)
        pltpu.make_async_copy(tbl.at[gidx_v.at[0]], rows_v.at[0], sem0).wait()
        pltpu.sync_copy(rows_v.at[0], acc.at[dst_v.at[b0]], add=True)

        @pl.when(k < NB // 2 - 1)
        def _():
            _gidx(0, b0 + 2, off, src_v, gidx_v)
            pltpu.async_copy(tbl.at[gidx_v.at[0]], rows_v.at[0], sem0)

        pltpu.make_async_copy(tbl.at[gidx_v.at[1]], rows_v.at[1], sem1).wait()
        pltpu.sync_copy(rows_v.at[1], acc.at[dst_v.at[b0 + 1]], add=True)
        return 0

    lax.fori_loop(0, NB // 2, body, 0)


def _sc_body(tbl, ps, pd, ns, nd, sump, sumn, cnts,
             src_v, dst_v, rows_v, gidx_v, ones_v, zb2, zb1, cbb, bb,
             acc, cacc, sem0, sem1):
    cid = lax.axis_index("c")
    sid = lax.axis_index("s")
    z16 = jnp.zeros((16,), jnp.float32)

    zbf = jnp.zeros((2, 16), jnp.bfloat16)

    def zb2body(i, _):
        zb2[pl.ds(2 * i, 2), :] = zbf
        return 0

    lax.fori_loop(0, EB // 2, zb2body, 0)
    _fill16(zb1, RPT, 0.0)
    _fill16(ones_v, EB, 1.0)

    for ei, (sref, dref, oref) in enumerate(((ps, pd, sump), (ns, nd, sumn))):
        pltpu.sync_copy(sref.at[pl.ds(sid * EPT, EPT)], src_v)
        pltpu.sync_copy(dref.at[sid], dst_v)

        @pl.when(cid == ei)
        def _():
            pltpu.sync_copy(zb1, cacc.at[pl.ds(sid * RPT, RPT)])
            plsc.subcore_barrier()

            def cbody(b, _):
                pltpu.sync_copy(ones_v, cacc.at[dst_v.at[b]], add=True)
                return 0

            lax.fori_loop(0, NB, cbody, 0)
            plsc.subcore_barrier()
            pltpu.sync_copy(cacc.at[pl.ds(sid * RPT, RPT)], cbb)
            pltpu.sync_copy(cbb, cnts.at[pl.ds(cid * NPAD + sid * RPT, RPT)])

        for jj in range(NCH // 2):
            chunk = 2 * jj + cid
            base = sid * RPT

            def zbody(i, _):
                pltpu.sync_copy(zb2, acc.at[pl.ds(base + i * EB, EB)])
                return 0

            lax.fori_loop(0, RPT // EB, zbody, 0)
            plsc.subcore_barrier()
            _scatter_job(tbl, src_v, dst_v, rows_v, gidx_v, acc,
                         chunk * N, sem0, sem1)
            plsc.subcore_barrier()
            for h in range(RPT // DR):
                pltpu.sync_copy(acc.at[pl.ds(base + h * DR, DR)], bb)
                pltpu.sync_copy(
                    bb, oref.at[pl.ds(chunk * NPAD + base + h * DR, DR)])
            plsc.subcore_barrier()


_segsum_sc = functools.partial(
    pl.kernel,
    out_type=[
        jax.ShapeDtypeStruct((NCH * NPAD, CH), jnp.bfloat16),
        jax.ShapeDtypeStruct((NCH * NPAD, CH), jnp.bfloat16),
        jax.ShapeDtypeStruct((2 * NPAD,), jnp.float32),
    ],
    mesh=plsc.VectorSubcoreMesh(core_axis_name="c", subcore_axis_name="s"),
    compiler_params=pltpu.CompilerParams(use_tc_tiling_on_sc=False),
    scratch_types=[
        pltpu.VMEM((EPT,), jnp.int32),
        pltpu.VMEM((NB, EB), jnp.int32),
        pltpu.VMEM((2, EB, CH), jnp.bfloat16),
        pltpu.VMEM((2, EB), jnp.int32),
        pltpu.VMEM((EB,), jnp.float32),
        pltpu.VMEM((EB, CH), jnp.bfloat16),
        pltpu.VMEM((RPT,), jnp.float32),
        pltpu.VMEM((RPT,), jnp.float32),
        pltpu.VMEM((DR, CH), jnp.bfloat16),
        pltpu.VMEM_SHARED((NPAD, CH), jnp.bfloat16),
        pltpu.VMEM_SHARED((NPAD,), jnp.float32),
        pltpu.SemaphoreType.DMA,
        pltpu.SemaphoreType.DMA,
    ],
)(_sc_body)


def _dense_body(sp_ref, sn_ref, x_ref, cp_ref, cn_ref, u_ref, v_ref, w_ref,
                b_ref, o_ref):
    rp = 1.0 / jnp.clip(cp_ref[...], 1.0, None)
    rn = 1.0 / jnp.clip(cn_ref[...], 1.0, None)
    sp = sp_ref[...].astype(jnp.float32) * rp
    sn = sn_ref[...].astype(jnp.float32) * rn
    acc = jnp.dot(sp, u_ref[...], preferred_element_type=jnp.float32)
    acc = acc + jnp.dot(sn, v_ref[...], preferred_element_type=jnp.float32)
    acc = acc + jnp.dot(x_ref[...], w_ref[...],
                        preferred_element_type=jnp.float32)
    o_ref[...] = jnp.maximum(acc + b_ref[...], 0.0)


def _dense_layer(sp, sn, x, cp, cn, u, v, w, b):
    bs = pl.BlockSpec((BN, H), lambda i: (i, 0))
    cs = pl.BlockSpec((BN, 1), lambda i: (i, 0))
    ws = pl.BlockSpec((H, H), lambda i: (0, 0))
    return pl.pallas_call(
        _dense_body,
        grid=(N // BN,),
        in_specs=[bs, bs, bs, cs, cs, ws, ws, ws,
                  pl.BlockSpec((1, H), lambda i: (0, 0))],
        out_specs=bs,
        out_shape=jax.ShapeDtypeStruct((N, H), jnp.float32),
    )(sp, sn, x, cp, cn, u, v, w, b)


def _chunked(feat):
    return (feat.astype(jnp.bfloat16)
            .reshape(N, NCH, CH).transpose(1, 0, 2).reshape(NCH * N, CH))


def _unchunk(flat):
    return (flat.reshape(NCH, NPAD, CH)[:, :N]
            .transpose(1, 0, 2).reshape(N, H))


def kernel(users_emb, items_emb, Wpl1, Wpr1, bpr1, Wnl1, Wnr1, bnr1,
           Wpl2, Wpr2, bpr2, Wnl2, Wnr2, bnr2,
           pos_edge_index, neg_edge_index):
    x = jnp.concatenate([users_emb, items_emb], axis=0)
    pad = E_PAD - E
    ps = jnp.concatenate([pos_edge_index[0].astype(jnp.int32),
                          jnp.zeros((pad,), jnp.int32)])
    pd = jnp.concatenate([pos_edge_index[1].astype(jnp.int32),
                          jnp.full((pad,), N, jnp.int32)]).reshape(NT, NB, EB)
    ns = jnp.concatenate([neg_edge_index[0].astype(jnp.int32),
                          jnp.zeros((pad,), jnp.int32)])
    nd = jnp.concatenate([neg_edge_index[1].astype(jnp.int32),
                          jnp.full((pad,), N, jnp.int32)]).reshape(NT, NB, EB)

    z64 = jnp.zeros((H // 2, H // 2), jnp.float32)
    u1 = jnp.concatenate([Wpl1, jnp.zeros_like(Wpl1)], axis=1)
    v1 = jnp.concatenate([jnp.zeros_like(Wnl1), Wnl1], axis=1)
    w1 = jnp.concatenate([Wpr1, Wnr1], axis=1)
    b1 = jnp.concatenate([bpr1, bnr1])[None, :]
    u2 = jnp.block([[Wpl2[:64], z64], [z64, Wnl2[:64]]])
    v2 = jnp.block([[z64, Wnl2[64:]], [Wpl2[64:], z64]])
    w2 = jnp.block([[Wpr2, z64], [z64, Wnr2]])
    b2 = jnp.concatenate([bpr2, bnr2])[None, :]

    sp1f, sn1f, cnts = _segsum_sc(_chunked(x), ps, pd, ns, nd)
    cp = cnts[:N][:, None]
    cn = cnts[NPAD:NPAD + N][:, None]
    z = _dense_layer(_unchunk(sp1f), _unchunk(sn1f), x, cp, cn,
                     u1, v1, w1, b1)

    sp2f, sn2f, _ = _segsum_sc(_chunked(z), ps, pd, ns, nd)
    out = _dense_layer(_unchunk(sp2f), _unchunk(sn2f), z, cp, cn,
                       u2, v2, w2, b2)
    return out


# speedup vs baseline: 4.1404x; 1.6104x over previous
import functools

import jax
import jax.numpy as jnp
from jax import lax
from jax.experimental import pallas as pl
from jax.experimental.pallas import tpu as pltpu
from jax.experimental.pallas import tpu_sc as plsc

N = 50000
H = 128
BN = 2000
NT = 16
NPAD = 50048
RPT = NPAD // NT
CH = 32
NCH = 4
EB = 128
E = 400000
EPT = E // NT
NBF = EPT // EB
TAIL = EPT - NBF * EB
NG = (NBF - 3) // 4
DB = 1024


def _fill16(ref, n, val):
    v = jnp.full((16,), val, jnp.float32)

    def body(i, _):
        ref[pl.ds(i * 16, 16)] = v
        return 0

    lax.fori_loop(0, n // 16, body, 0)


def _sc_body(tbl, ps, pd, ns, nd, sump, sumn, cnts,
             src_v, dst_v, rows_v, gidx_v, didx_v, ones_v, zb2, bb,
             acc, gs0, gs1, gs2, gs3, ss0, ss1, ss2, ss3):
    cid = lax.axis_index("c")
    sid = lax.axis_index("s")
    gs = (gs0, gs1, gs2, gs3)
    ss = (ss0, ss1, ss2, ss3)

    def build(slot, b, off):
        for kk in range(EB // 16):
            sv = src_v[pl.ds(b * EB + 16 * kk, 16)]
            gidx_v[slot, pl.ds(16 * kk, 16)] = sv + off
            dv = dst_v[pl.ds(b * EB + 16 * kk, 16)]
            didx_v[slot, pl.ds(16 * kk, 16)] = dv

    def build_tail(slot, off):
        for kk in range(EB // 16):
            valid = lax.iota(jnp.int32, 16) + (16 * kk) < TAIL
            sv = src_v[pl.ds(NBF * EB + 16 * kk, 16)]
            gidx_v[slot, pl.ds(16 * kk, 16)] = jnp.where(valid, sv + off, 0)
            dv = dst_v[pl.ds(NBF * EB + 16 * kk, 16)]
            didx_v[slot, pl.ds(16 * kk, 16)] = jnp.where(valid, dv, N)

    def build_cnt(slot, b):
        for kk in range(EB // 16):
            dv = dst_v[pl.ds(b * EB + 16 * kk, 16)]
            didx_v[slot, pl.ds(16 * kk, 16)] = dv

    def build_cnt_tail(slot):
        for kk in range(EB // 16):
            valid = lax.iota(jnp.int32, 16) + (16 * kk) < TAIL
            dv = dst_v[pl.ds(NBF * EB + 16 * kk, 16)]
            didx_v[slot, pl.ds(16 * kk, 16)] = jnp.where(valid, dv, N)

    def g_start(slot):
        pltpu.async_copy(tbl.at[gidx_v.at[slot]], rows_v.at[slot], gs[slot])

    def g_wait(slot):
        pltpu.make_async_copy(
            tbl.at[gidx_v.at[slot]], rows_v.at[slot], gs[slot]).wait()

    def s_start(slot):
        pltpu.async_copy(rows_v.at[slot], acc.at[didx_v.at[slot]], ss[slot],
                         add=True)

    def s_wait(slot):
        pltpu.make_async_copy(
            rows_v.at[slot], acc.at[didx_v.at[slot]], ss[slot]).wait()

    def scatter_job(off):
        build(0, 0, off)
        g_start(0)
        build(1, 1, off)
        g_start(1)

        def body(m, _):
            for j in range(4):
                b = 4 * m + j
                g_wait(j)
                if j < 2:
                    @pl.when(b >= 2)
                    def _():
                        s_wait((j + 2) % 4)
                else:
                    s_wait((j + 2) % 4)
                s_start(j)
                build((j + 2) % 4, b + 2, off)
                g_start((j + 2) % 4)
            return 0

        lax.fori_loop(0, NG, body, 0)
        g_wait(0)
        s_wait(2)
        s_start(0)
        build(2, NBF - 1, off)
        g_start(2)
        g_wait(1)
        s_wait(3)
        s_start(1)
        build_tail(3, off)
        g_start(3)
        g_wait(2)
        s_start(2)
        g_wait(3)
        s_start(3)
        for j in range(4):
            s_wait(j)

    def c_start(slot):
        pltpu.async_copy(ones_v, acc.at[didx_v.at[slot]], ss[slot], add=True)

    def c_wait(slot):
        pltpu.make_async_copy(
            ones_v, acc.at[didx_v.at[slot]], ss[slot]).wait()

    def counts_job():
        def body(k, _):
            for j in range(2):
                b = 2 * k + j

                @pl.when(k > 0)
                def _():
                    c_wait(j)

                build_cnt(j, b)
                c_start(j)
            return 0

        lax.fori_loop(0, NBF // 2, body, 0)
        c_wait(0)
        build_cnt(0, NBF - 1)
        c_start(0)
        c_wait(1)
        build_cnt_tail(1)
        c_start(1)
        c_wait(0)
        c_wait(1)

    zrow = jnp.zeros((CH,), jnp.bfloat16)
    orow = jnp.ones((CH,), jnp.bfloat16)

    def fillbody(i, _):
        zb2[i, pl.ds(0, CH)] = zrow
        ones_v[i, pl.ds(0, CH)] = orow
        return 0

    lax.fori_loop(0, EB, fillbody, 0)

    base = sid * RPT

    def zero_acc():
        def zbody(i, _):
            pltpu.sync_copy(zb2, acc.at[pl.ds(base + i * EB, EB)])
            return 0

        lax.fori_loop(0, RPT // EB, zbody, 0)
        pltpu.sync_copy(zb2.at[pl.ds(0, RPT - (RPT // EB) * EB)],
                        acc.at[pl.ds(base + (RPT // EB) * EB,
                                     RPT - (RPT // EB) * EB)])

    def drain_acc(oref, row0):
        for h in range(RPT // DB):
            pltpu.sync_copy(acc.at[pl.ds(base + h * DB, DB)], bb)
            pltpu.sync_copy(bb, oref.at[pl.ds(row0 + base + h * DB, DB)])
        rem = RPT - (RPT // DB) * DB
        pltpu.sync_copy(acc.at[pl.ds(base + RPT - rem, rem)],
                        bb.at[pl.ds(0, rem)])
        pltpu.sync_copy(bb.at[pl.ds(0, rem)],
                        oref.at[pl.ds(row0 + base + RPT - rem, rem)])
    for ei, (sref, dref, oref) in enumerate(((ps, pd, sump), (ns, nd, sumn))):
        pltpu.sync_copy(sref.at[pl.ds(sid * EPT, EPT)], src_v.at[pl.ds(0, EPT)])
        pltpu.sync_copy(dref.at[pl.ds(sid * EPT, EPT)], dst_v.at[pl.ds(0, EPT)])

        @pl.when(cid == ei)
        def _():
            zero_acc()
            plsc.subcore_barrier()
            counts_job()
            plsc.subcore_barrier()
            drain_acc(cnts, ei * NPAD)
            plsc.subcore_barrier()

        for jj in range(NCH // 2):
            chunk = 2 * jj + cid
            zero_acc()
            plsc.subcore_barrier()
            scatter_job(chunk * N)
            plsc.subcore_barrier()
            drain_acc(oref, chunk * NPAD)
            plsc.subcore_barrier()


_segsum_sc = functools.partial(
    pl.kernel,
    out_type=[
        jax.ShapeDtypeStruct((NCH * NPAD, CH), jnp.bfloat16),
        jax.ShapeDtypeStruct((NCH * NPAD, CH), jnp.bfloat16),
        jax.ShapeDtypeStruct((2 * NPAD, CH), jnp.bfloat16),
    ],
    mesh=plsc.VectorSubcoreMesh(core_axis_name="c", subcore_axis_name="s"),
    compiler_params=pltpu.CompilerParams(use_tc_tiling_on_sc=False),
    scratch_types=[
        pltpu.VMEM((EPT + EB, ), jnp.int32),
        pltpu.VMEM((EPT + EB, ), jnp.int32),
        pltpu.VMEM((4, EB, CH), jnp.bfloat16),
        pltpu.VMEM((4, EB), jnp.int32),
        pltpu.VMEM((4, EB), jnp.int32),
        pltpu.VMEM((EB, CH), jnp.bfloat16),
        pltpu.VMEM((EB, CH), jnp.bfloat16),
        pltpu.VMEM((DB, CH), jnp.bfloat16),
        pltpu.VMEM_SHARED((NPAD, CH), jnp.bfloat16),
        pltpu.SemaphoreType.DMA,
        pltpu.SemaphoreType.DMA,
        pltpu.SemaphoreType.DMA,
        pltpu.SemaphoreType.DMA,
        pltpu.SemaphoreType.DMA,
        pltpu.SemaphoreType.DMA,
        pltpu.SemaphoreType.DMA,
        pltpu.SemaphoreType.DMA,
    ],
)(_sc_body)


def _dense_body(sp_ref, sn_ref, x_ref, cp_ref, cn_ref, u_ref, v_ref, w_ref,
                b_ref, o_ref):
    rp = 1.0 / jnp.clip(cp_ref[...], 1.0, None)
    rn = 1.0 / jnp.clip(cn_ref[...], 1.0, None)
    sp = sp_ref[...].astype(jnp.float32) * rp
    sn = sn_ref[...].astype(jnp.float32) * rn
    acc = jnp.dot(sp, u_ref[...], preferred_element_type=jnp.float32)
    acc = acc + jnp.dot(sn, v_ref[...], preferred_element_type=jnp.float32)
    acc = acc + jnp.dot(x_ref[...], w_ref[...],
                        preferred_element_type=jnp.float32)
    o_ref[...] = jnp.maximum(acc + b_ref[...], 0.0)


def _dense_layer(sp, sn, x, cp, cn, u, v, w, b):
    bs = pl.BlockSpec((BN, H), lambda i: (i, 0))
    cs = pl.BlockSpec((BN, 1), lambda i: (i, 0))
    ws = pl.BlockSpec((H, H), lambda i: (0, 0))
    return pl.pallas_call(
        _dense_body,
        grid=(N // BN,),
        in_specs=[bs, bs, bs, cs, cs, ws, ws, ws,
                  pl.BlockSpec((1, H), lambda i: (0, 0))],
        out_specs=bs,
        out_shape=jax.ShapeDtypeStruct((N, H), jnp.float32),
    )(sp, sn, x, cp, cn, u, v, w, b)


def _chunked(feat):
    return (feat.astype(jnp.bfloat16)
            .reshape(N, NCH, CH).transpose(1, 0, 2).reshape(NCH * N, CH))


def _unchunk(flat):
    return (flat.reshape(NCH, NPAD, CH)[:, :N]
            .transpose(1, 0, 2).reshape(N, H))


def kernel(users_emb, items_emb, Wpl1, Wpr1, bpr1, Wnl1, Wnr1, bnr1,
           Wpl2, Wpr2, bpr2, Wnl2, Wnr2, bnr2,
           pos_edge_index, neg_edge_index):
    x = jnp.concatenate([users_emb, items_emb], axis=0)
    ps = pos_edge_index[0].astype(jnp.int32)
    pd = pos_edge_index[1].astype(jnp.int32)
    ns = neg_edge_index[0].astype(jnp.int32)
    nd = neg_edge_index[1].astype(jnp.int32)

    z64 = jnp.zeros((H // 2, H // 2), jnp.float32)
    u1 = jnp.concatenate([Wpl1, jnp.zeros_like(Wpl1)], axis=1)
    v1 = jnp.concatenate([jnp.zeros_like(Wnl1), Wnl1], axis=1)
    w1 = jnp.concatenate([Wpr1, Wnr1], axis=1)
    b1 = jnp.concatenate([bpr1, bnr1])[None, :]
    u2 = jnp.block([[Wpl2[:64], z64], [z64, Wnl2[:64]]])
    v2 = jnp.block([[z64, Wnl2[64:]], [Wpl2[64:], z64]])
    w2 = jnp.block([[Wpr2, z64], [z64, Wnr2]])
    b2 = jnp.concatenate([bpr2, bnr2])[None, :]

    sp1f, sn1f, cnts = _segsum_sc(_chunked(x), ps, pd, ns, nd)
    cp = cnts[:N, :1]
    cn = cnts[NPAD:NPAD + N, :1]
    z = _dense_layer(_unchunk(sp1f), _unchunk(sn1f), x, cp, cn,
                     u1, v1, w1, b1)

    sp2f, sn2f, _ = _segsum_sc(_chunked(z), ps, pd, ns, nd)
    out = _dense_layer(_unchunk(sp2f), _unchunk(sn2f), z, cp, cn,
                       u2, v2, w2, b2)
    return out


# speedup vs baseline: 4.7737x; 1.1530x over previous
import functools

import jax
import jax.numpy as jnp
from jax import lax
from jax.experimental import pallas as pl
from jax.experimental.pallas import tpu as pltpu
from jax.experimental.pallas import tpu_sc as plsc

N = 50000
NU = 30000
H = 128
BN = 2000
NT = 16
NPAD = 50048
RPT = NPAD // NT
CH = 32
NCH = 4
EB = 128
E = 400000
EPT = E // NT
NBF = EPT // EB
TAIL = EPT - NBF * EB
NG = (NBF - 3) // 4
DB = 1024


def _sc_body(with_counts, *refs):
    if with_counts:
        (tbl, ps, pd, ns, nd, sump, sumn, cnts,
         src_v, dst_v, rows_v, gidx_v, didx_v, ones_v, zb2, bb, acc,
         gs0, gs1, gs2, gs3, ss0, ss1, ss2, ss3) = refs
    else:
        (tbl, ps, pd, ns, nd, sump, sumn,
         src_v, dst_v, rows_v, gidx_v, didx_v, ones_v, zb2, bb, acc,
         gs0, gs1, gs2, gs3, ss0, ss1, ss2, ss3) = refs
    cid = lax.axis_index("c")
    sid = lax.axis_index("s")
    gs = (gs0, gs1, gs2, gs3)
    ss = (ss0, ss1, ss2, ss3)

    def build(slot, b, off):
        for kk in range(EB // 16):
            sv = src_v[pl.ds(b * EB + 16 * kk, 16)]
            gidx_v[slot, pl.ds(16 * kk, 16)] = sv + off
            dv = dst_v[pl.ds(b * EB + 16 * kk, 16)]
            didx_v[slot, pl.ds(16 * kk, 16)] = dv

    def build_tail(slot, off):
        for kk in range(EB // 16):
            valid = lax.iota(jnp.int32, 16) + (16 * kk) < TAIL
            sv = src_v[pl.ds(NBF * EB + 16 * kk, 16)]
            gidx_v[slot, pl.ds(16 * kk, 16)] = jnp.where(valid, sv + off, 0)
            dv = dst_v[pl.ds(NBF * EB + 16 * kk, 16)]
            didx_v[slot, pl.ds(16 * kk, 16)] = jnp.where(valid, dv, N)

    def build_cnt(slot, b):
        for kk in range(EB // 16):
            dv = dst_v[pl.ds(b * EB + 16 * kk, 16)]
            didx_v[slot, pl.ds(16 * kk, 16)] = dv

    def build_cnt_tail(slot):
        for kk in range(EB // 16):
            valid = lax.iota(jnp.int32, 16) + (16 * kk) < TAIL
            dv = dst_v[pl.ds(NBF * EB + 16 * kk, 16)]
            didx_v[slot, pl.ds(16 * kk, 16)] = jnp.where(valid, dv, N)

    def g_start(slot):
        pltpu.async_copy(tbl.at[gidx_v.at[slot]], rows_v.at[slot], gs[slot])

    def g_wait(slot):
        pltpu.make_async_copy(
            tbl.at[gidx_v.at[slot]], rows_v.at[slot], gs[slot]).wait()

    def s_start(slot):
        pltpu.async_copy(rows_v.at[slot], acc.at[didx_v.at[slot]], ss[slot],
                         add=True)

    def s_wait(slot):
        pltpu.make_async_copy(
            rows_v.at[slot], acc.at[didx_v.at[slot]], ss[slot]).wait()

    def scatter_job(off):
        build(0, 0, off)
        g_start(0)
        build(1, 1, off)
        g_start(1)

        def body(m, _):
            for j in range(4):
                b = 4 * m + j
                g_wait(j)
                if j < 2:
                    @pl.when(b >= 2)
                    def _():
                        s_wait((j + 2) % 4)
                else:
                    s_wait((j + 2) % 4)
                s_start(j)
                build((j + 2) % 4, b + 2, off)
                g_start((j + 2) % 4)
            return 0

        lax.fori_loop(0, NG, body, 0)
        g_wait(0)
        s_wait(2)
        s_start(0)
        build(2, NBF - 1, off)
        g_start(2)
        g_wait(1)
        s_wait(3)
        s_start(1)
        build_tail(3, off)
        g_start(3)
        g_wait(2)
        s_start(2)
        g_wait(3)
        s_start(3)
        for j in range(4):
            s_wait(j)

    def c_start(slot):
        pltpu.async_copy(ones_v, acc.at[didx_v.at[slot]], ss[slot], add=True)

    def c_wait(slot):
        pltpu.make_async_copy(
            ones_v, acc.at[didx_v.at[slot]], ss[slot]).wait()

    def counts_job():
        def body(k, _):
            for j in range(2):
                b = 2 * k + j

                @pl.when(k > 0)
                def _():
                    c_wait(j)

                build_cnt(j, b)
                c_start(j)
            return 0

        lax.fori_loop(0, NBF // 2, body, 0)
        c_wait(0)
        build_cnt(0, NBF - 1)
        c_start(0)
        c_wait(1)
        build_cnt_tail(1)
        c_start(1)
        c_wait(0)
        c_wait(1)

    zrow = jnp.zeros((CH,), jnp.bfloat16)
    orow = jnp.ones((CH,), jnp.bfloat16)

    def fillbody(i, _):
        zb2[i, pl.ds(0, CH)] = zrow
        ones_v[i, pl.ds(0, CH)] = orow
        return 0

    lax.fori_loop(0, EB, fillbody, 0)

    base = sid * RPT

    def zero_acc():
        def zbody(i, _):
            pltpu.sync_copy(zb2, acc.at[pl.ds(base + i * EB, EB)])
            return 0

        lax.fori_loop(0, RPT // EB, zbody, 0)
        pltpu.sync_copy(zb2.at[pl.ds(0, RPT - (RPT // EB) * EB)],
                        acc.at[pl.ds(base + (RPT // EB) * EB,
                                     RPT - (RPT // EB) * EB)])

    def drain_acc(oref, row0):
        for h in range(RPT // DB):
            pltpu.sync_copy(acc.at[pl.ds(base + h * DB, DB)], bb)
            pltpu.sync_copy(bb, oref.at[pl.ds(row0 + base + h * DB, DB)])
        rem = RPT - (RPT // DB) * DB
        pltpu.sync_copy(acc.at[pl.ds(base + RPT - rem, rem)],
                        bb.at[pl.ds(0, rem)])
        pltpu.sync_copy(bb.at[pl.ds(0, rem)],
                        oref.at[pl.ds(row0 + base + RPT - rem, rem)])

    for ei, (sref, dref, oref) in enumerate(((ps, pd, sump), (ns, nd, sumn))):
        pltpu.sync_copy(sref.at[pl.ds(sid * EPT, EPT)],
                        src_v.at[pl.ds(0, EPT)])
        pltpu.sync_copy(dref.at[pl.ds(sid * EPT, EPT)],
                        dst_v.at[pl.ds(0, EPT)])

        if with_counts:
            @pl.when(cid == ei)
            def _():
                zero_acc()
                plsc.subcore_barrier()
                counts_job()
                plsc.subcore_barrier()
                drain_acc(cnts, ei * NPAD)
                plsc.subcore_barrier()

        for jj in range(NCH // 2):
            chunk = 2 * jj + cid
            zero_acc()
            plsc.subcore_barrier()
            scatter_job(chunk * N)
            plsc.subcore_barrier()
            drain_acc(oref, chunk * NPAD)
            plsc.subcore_barrier()


def _make_segsum(with_counts):
    out_type = [
        jax.ShapeDtypeStruct((NCH * NPAD, CH), jnp.bfloat16),
        jax.ShapeDtypeStruct((NCH * NPAD, CH), jnp.bfloat16),
    ]
    if with_counts:
        out_type.append(jax.ShapeDtypeStruct((2 * NPAD, CH), jnp.bfloat16))
    return functools.partial(
        pl.kernel,
        out_type=out_type,
        mesh=plsc.VectorSubcoreMesh(core_axis_name="c", subcore_axis_name="s"),
        compiler_params=pltpu.CompilerParams(use_tc_tiling_on_sc=False),
        scratch_types=[
            pltpu.VMEM((EPT + EB, ), jnp.int32),
            pltpu.VMEM((EPT + EB, ), jnp.int32),
            pltpu.VMEM((4, EB, CH), jnp.bfloat16),
            pltpu.VMEM((4, EB), jnp.int32),
            pltpu.VMEM((4, EB), jnp.int32),
            pltpu.VMEM((EB, CH), jnp.bfloat16),
            pltpu.VMEM((EB, CH), jnp.bfloat16),
            pltpu.VMEM((DB, CH), jnp.bfloat16),
            pltpu.VMEM_SHARED((NPAD, CH), jnp.bfloat16),
            pltpu.SemaphoreType.DMA,
            pltpu.SemaphoreType.DMA,
            pltpu.SemaphoreType.DMA,
            pltpu.SemaphoreType.DMA,
            pltpu.SemaphoreType.DMA,
            pltpu.SemaphoreType.DMA,
            pltpu.SemaphoreType.DMA,
            pltpu.SemaphoreType.DMA,
        ],
    )(functools.partial(_sc_body, with_counts))


_segsum_cnt = _make_segsum(True)
_segsum_nc = _make_segsum(False)


def _chunk_body(u_ref, it_ref, x_ref, t_ref):
    i = pl.program_id(0)
    v = jnp.where(i < NU // BN, u_ref[...], it_ref[...])
    x_ref[...] = v
    for c in range(NCH):
        t_ref[c] = v[:, c * CH:(c + 1) * CH].astype(jnp.bfloat16)


def _chunker(users, items):
    return pl.pallas_call(
        _chunk_body,
        grid=(N // BN,),
        in_specs=[
            pl.BlockSpec((BN, H), lambda i: (jnp.minimum(i, NU // BN - 1), 0)),
            pl.BlockSpec((BN, H), lambda i: (jnp.maximum(i - NU // BN, 0), 0)),
        ],
        out_specs=[
            pl.BlockSpec((BN, H), lambda i: (i, 0)),
            pl.BlockSpec((NCH, BN, CH), lambda i: (0, i, 0)),
        ],
        out_shape=[
            jax.ShapeDtypeStruct((N, H), jnp.float32),
            jax.ShapeDtypeStruct((NCH, N, CH), jnp.bfloat16),
        ],
    )(users, items)


def _dense_body(emit_table, spc_ref, snc_ref, x_ref, cp_ref, cn_ref,
                u_ref, v_ref, w_ref, b_ref, *outs):
    sp = jnp.concatenate([spc_ref[c] for c in range(NCH)],
                         axis=-1).astype(jnp.float32)
    sn = jnp.concatenate([snc_ref[c] for c in range(NCH)],
                         axis=-1).astype(jnp.float32)
    rp = 1.0 / jnp.clip(cp_ref[...].astype(jnp.float32), 1.0, None)
    rn = 1.0 / jnp.clip(cn_ref[...].astype(jnp.float32), 1.0, None)
    acc = jnp.dot(sp * rp, u_ref[...], preferred_element_type=jnp.float32)
    acc = acc + jnp.dot(sn * rn, v_ref[...],
                        preferred_element_type=jnp.float32)
    acc = acc + jnp.dot(x_ref[...], w_ref[...],
                        preferred_element_type=jnp.float32)
    z = jnp.maximum(acc + b_ref[...], 0.0)
    outs[0][...] = z
    if emit_table:
        for c in range(NCH):
            outs[1][c] = z[:, c * CH:(c + 1) * CH].astype(jnp.bfloat16)


def _dense_layer(emit_table, spc, snc, x, cp, cn, u, v, w, b):
    bs = pl.BlockSpec((BN, H), lambda i: (i, 0))
    cs = pl.BlockSpec((BN, 1), lambda i: (i, 0))
    ws = pl.BlockSpec((H, H), lambda i: (0, 0))
    sums = pl.BlockSpec((NCH, BN, CH), lambda i: (0, i, 0))
    out_specs = [bs]
    out_shape = [jax.ShapeDtypeStruct((N, H), jnp.float32)]
    if emit_table:
        out_specs.append(pl.BlockSpec((NCH, BN, CH), lambda i: (0, i, 0)))
        out_shape.append(jax.ShapeDtypeStruct((NCH, N, CH), jnp.bfloat16))
    return pl.pallas_call(
        functools.partial(_dense_body, emit_table),
        grid=(N // BN,),
        in_specs=[sums, sums, bs, cs, cs, ws, ws, ws,
                  pl.BlockSpec((1, H), lambda i: (0, 0))],
        out_specs=out_specs,
        out_shape=out_shape,
    )(spc, snc, x, cp, cn, u, v, w, b)


def kernel(users_emb, items_emb, Wpl1, Wpr1, bpr1, Wnl1, Wnr1, bnr1,
           Wpl2, Wpr2, bpr2, Wnl2, Wnr2, bnr2,
           pos_edge_index, neg_edge_index):
    ps = pos_edge_index[0].astype(jnp.int32)
    pd = pos_edge_index[1].astype(jnp.int32)
    ns = neg_edge_index[0].astype(jnp.int32)
    nd = neg_edge_index[1].astype(jnp.int32)

    z64 = jnp.zeros((H // 2, H // 2), jnp.float32)
    u1 = jnp.concatenate([Wpl1, jnp.zeros_like(Wpl1)], axis=1)
    v1 = jnp.concatenate([jnp.zeros_like(Wnl1), Wnl1], axis=1)
    w1 = jnp.concatenate([Wpr1, Wnr1], axis=1)
    b1 = jnp.concatenate([bpr1, bnr1])[None, :]
    u2 = jnp.block([[Wpl2[:64], z64], [z64, Wnl2[:64]]])
    v2 = jnp.block([[z64, Wnl2[64:]], [Wpl2[64:], z64]])
    w2 = jnp.block([[Wpr2, z64], [z64, Wnr2]])
    b2 = jnp.concatenate([bpr2, bnr2])[None, :]

    x, xt = _chunker(users_emb, items_emb)
    sp1f, sn1f, cnts = _segsum_cnt(xt.reshape(NCH * N, CH), ps, pd, ns, nd)
    cp = cnts[:N, :1]
    cn = cnts[NPAD:NPAD + N, :1]
    z, zt = _dense_layer(True, sp1f.reshape(NCH, NPAD, CH),
                         sn1f.reshape(NCH, NPAD, CH), x, cp, cn,
                         u1, v1, w1, b1)

    sp2f, sn2f = _segsum_nc(zt.reshape(NCH * N, CH), ps, pd, ns, nd)
    (out,) = _dense_layer(False, sp2f.reshape(NCH, NPAD, CH),
                          sn2f.reshape(NCH, NPAD, CH), z, cp, cn,
                          u2, v2, w2, b2)
    return out
